# baseline (speedup 1.0000x reference)
import functools
import math

import jax
import jax.numpy as jnp
from jax import lax
from jax.experimental import pallas as pl
from jax.experimental.pallas import tpu as pltpu
from jax.experimental.pallas import tpu_sc as plsc

_N = 10000
_E = 320000
_D = 128
_ALPHA = 0.1
_BETA = math.log(1.0 / 2.0 + 1.0)

_NC = 2
_NS = 16
_NW = _NC * _NS
_CH = 40
_NCH = _E // _CH
_CPT = _NCH // _NW
_NB = 5
_NP = 10240
_RPT = _NP // _NS
_ZRD = 128
_ZRA = 16
_CPP = _NCH // _NS
_GRW = _NP // _NW
_FT = 80
_CHA = 80
_NCHA = _E // _CHA
_CPA = _NCHA // _NW
_NBA = 2

_mesh = plsc.VectorSubcoreMesh(core_axis_name="c", subcore_axis_name="s")
_sc_params = pltpu.CompilerParams(use_tc_tiling_on_sc=False)


def _prep_body(dst2_hbm, feat_hbm, g_hbm, normb_hbm,
               shared, idx_v, ones_v, zer_v, dbuf, nbuf, fbuf, gbuf,
               s0, s1, s2, s3, s4):
    cid = lax.axis_index("c")
    sid = lax.axis_index("s")
    ssem = (s0, s1, s2, s3, s4)

    @pl.loop(0, _ZRD)
    def _(i):
        zer_v[i] = jnp.zeros((16,), jnp.float32)

    @pl.loop(0, _CH)
    def _(i):
        ones_v[i] = jnp.ones((16,), jnp.float32)

    rbase = sid * _RPT

    @pl.loop(0, _RPT // _ZRD)
    def _(j):
        pltpu.sync_copy(zer_v, shared.at[pl.ds(rbase + j * _ZRD, _ZRD)])

    plsc.subcore_barrier()

    crow = sid * _CPP
    pltpu.sync_copy(dst2_hbm.at[pl.ds(crow, _CPP)], idx_v)

    @pl.loop(0, _CPP // _NB)
    def _(g):
        for b in range(_NB):
            @pl.when(g > 0)
            def _():
                pltpu.make_async_copy(
                    ones_v, shared.at[idx_v.at[0]], ssem[b]).wait()

            pltpu.async_copy(
                ones_v, shared.at[idx_v.at[g * _NB + b]], ssem[b], add=True)

    for b in range(_NB):
        pltpu.make_async_copy(ones_v, shared.at[idx_v.at[0]], ssem[b]).wait()

    plsc.subcore_barrier()

    base = (cid * _NS + sid) * _GRW
    pltpu.sync_copy(shared.at[pl.ds(base, _GRW)], dbuf)

    @pl.loop(0, _GRW)
    def _(r):
        d = jnp.maximum(dbuf[r], 1.0)
        i = lax.bitcast_convert_type(d, jnp.int32)
        i = jnp.int32(0x5F3759DF) - lax.shift_right_logical(i, 1)
        y = lax.bitcast_convert_type(i, jnp.float32)
        y = y * (1.5 - 0.5 * d * y * y)
        y = y * (1.5 - 0.5 * d * y * y)
        y = y * (1.5 - 0.5 * d * y * y)
        nbuf[r] = y

    pltpu.sync_copy(nbuf, normb_hbm.at[pl.ds(base, _GRW)])

    for t in range(_GRW // _FT):
        row0 = base + t * _FT

        @pl.when(row0 < _N)
        def _():
            pltpu.sync_copy(feat_hbm.at[pl.ds(row0, _FT)], fbuf)

            @pl.loop(0, _FT)
            def _(r):
                nv = nbuf[t * _FT + r]
                for j in range(_D // 16):
                    gbuf[r, pl.ds(j * 16, 16)] = (
                        fbuf[r, pl.ds(j * 16, 16)] * nv)

            pltpu.sync_copy(gbuf, g_hbm.at[pl.ds(row0, _FT)])


def _agg_body(g_hbm, src2_hbm, dst2_hbm, aggp_hbm,
              shared, sidx_v, didx_v, r0, r1, zer_v,
              g0, g1, t0, t1):
    cid = lax.axis_index("c")
    sid = lax.axis_index("s")
    rows = (r0, r1)
    gsem = (g0, g1)
    ssem = (t0, t1)

    @pl.loop(0, _ZRA)
    def _(i):
        @pl.loop(0, _D // 16)
        def _(j):
            zer_v[i, pl.ds(j * 16, 16)] = jnp.zeros((16,), jnp.float32)

    rbase = sid * _RPT

    @pl.loop(0, _RPT // _ZRA)
    def _(j):
        pltpu.sync_copy(zer_v, shared.at[pl.ds(rbase + j * _ZRA, _ZRA)])

    plsc.subcore_barrier()

    crow = (sid * _NC + cid) * _CPA
    pltpu.sync_copy(src2_hbm.at[pl.ds(crow, _CPA)], sidx_v)
    pltpu.sync_copy(dst2_hbm.at[pl.ds(crow, _CPA)], didx_v)

    def issue_gather(t, b):
        pltpu.async_copy(g_hbm.at[sidx_v.at[t]], rows[b], gsem[b])

    def wait_gather(t, b):
        pltpu.make_async_copy(g_hbm.at[sidx_v.at[t]], rows[b], gsem[b]).wait()

    def issue_scatter(t, b):
        pltpu.async_copy(rows[b], shared.at[didx_v.at[t]], ssem[b], add=True)

    def wait_scatter(b):
        pltpu.make_async_copy(rows[b], shared.at[didx_v.at[0]], ssem[b]).wait()

    issue_gather(0, 0)
    issue_gather(1, 1)

    @pl.loop(0, (_CPA - 1) // _NBA)
    def _(gg):
        for b in range(_NBA):
            u = gg * _NBA + b
            wait_gather(u, b)
            issue_scatter(u, b)

            @pl.when(u <= _CPA - 3)
            def _():
                wait_scatter(b)
                issue_gather(u + _NBA, b)

    wait_gather(_CPA - 1, (_CPA - 1) % _NBA)
    issue_scatter(_CPA - 1, (_CPA - 1) % _NBA)
    for b in range(_NBA):
        wait_scatter(b)

    plsc.subcore_barrier()
    pltpu.sync_copy(shared.at[pl.ds(rbase, _RPT)],
                    aggp_hbm.at[cid].at[pl.ds(rbase, _RPT)])


def _final_body(aggp_ref, normb_ref, f0_ref, w_ref, b_ref, out_ref):
    norm = normb_ref[0:_N, 0:1]
    agg = aggp_ref[0, 0:_N, :] + aggp_ref[1, 0:_N, :]
    rst0 = agg * norm * (1.0 - _ALPHA) + _ALPHA * f0_ref[...]
    rst = (1.0 - _BETA) * rst0 + _BETA * jnp.dot(
        rst0, w_ref[...], preferred_element_type=jnp.float32)
    out_ref[...] = rst + b_ref[...]


_prep_call = functools.partial(
    pl.kernel,
    out_type=(
        jax.ShapeDtypeStruct((_NP, _D), jnp.float32),
        jax.ShapeDtypeStruct((_NP, 16), jnp.float32),
    ),
    mesh=_mesh,
    compiler_params=_sc_params,
    scratch_types=[
        pltpu.VMEM_SHARED((_NP, 16), jnp.float32),
        pltpu.VMEM((_CPP, _CH), jnp.int32),
        pltpu.VMEM((_CH, 16), jnp.float32),
        pltpu.VMEM((_ZRD, 16), jnp.float32),
        pltpu.VMEM((_GRW, 16), jnp.float32),
        pltpu.VMEM((_GRW, 16), jnp.float32),
        pltpu.VMEM((_FT, _D), jnp.float32),
        pltpu.VMEM((_FT, _D), jnp.float32),
    ] + [pltpu.SemaphoreType.DMA] * _NB,
)(_prep_body)


_agg_call = functools.partial(
    pl.kernel,
    out_type=jax.ShapeDtypeStruct((_NC, _NP, _D), jnp.float32),
    mesh=_mesh,
    compiler_params=_sc_params,
    scratch_types=[
        pltpu.VMEM_SHARED((_NP, _D), jnp.float32),
        pltpu.VMEM((_CPA, _CHA), jnp.int32),
        pltpu.VMEM((_CPA, _CHA), jnp.int32),
    ] + [pltpu.VMEM((_CHA, _D), jnp.float32)] * _NBA + [
        pltpu.VMEM((_ZRA, _D), jnp.float32),
    ] + [pltpu.SemaphoreType.DMA] * (2 * _NBA),
)(_agg_body)


def kernel(feat, feat_0, edge_index, weight1, bias):
    dst2 = edge_index[1].reshape(_NCH, _CH)
    src2a = edge_index[0].reshape(_NCHA, _CHA)
    dst2a = edge_index[1].reshape(_NCHA, _CHA)

    g, normb = _prep_call(dst2, feat)
    aggp = _agg_call(g, src2a, dst2a)

    out = pl.pallas_call(
        _final_body,
        out_shape=jax.ShapeDtypeStruct((_N, _D), jnp.float32),
    )(aggp, normb, feat_0, weight1, bias.reshape(1, _D))
    return out


# speedup vs baseline: 1.1156x; 1.1156x over previous
import functools
import math

import jax
import jax.numpy as jnp
from jax import lax
from jax.experimental import pallas as pl
from jax.experimental.pallas import tpu as pltpu
from jax.experimental.pallas import tpu_sc as plsc

_N = 10000
_E = 320000
_D = 128
_ALPHA = 0.1
_BETA = math.log(1.0 / 2.0 + 1.0)

_NC = 2
_NS = 16
_NW = _NC * _NS
_CH = 40
_NCH = _E // _CH
_CPT = _NCH // _NW
_NB = 5
_NP = 10240
_RPT = _NP // _NS
_ZRD = 128
_ZRA = 16
_CPP = _NCH // _NS
_GRW = _NP // _NW
_FT = 80
_CHA = 80
_NCHA = _E // _CHA
_CPA = _NCHA // _NW
_NBA = 2

_mesh = plsc.VectorSubcoreMesh(core_axis_name="c", subcore_axis_name="s")
_sc_params = pltpu.CompilerParams(use_tc_tiling_on_sc=False)


def _prep_body(dst2_hbm, feat_hbm, g_hbm, normb_hbm,
               shared, idx_v, ones_v, zer_v, dbuf, nbuf, fbuf, gbuf,
               s0, s1, s2, s3, s4):
    cid = lax.axis_index("c")
    sid = lax.axis_index("s")
    ssem = (s0, s1, s2, s3, s4)

    @pl.loop(0, _ZRD)
    def _(i):
        zer_v[i] = jnp.zeros((16,), jnp.float32)

    @pl.loop(0, _CH)
    def _(i):
        ones_v[i] = jnp.ones((16,), jnp.float32)

    rbase = sid * _RPT

    @pl.loop(0, _RPT // _ZRD)
    def _(j):
        pltpu.sync_copy(zer_v, shared.at[pl.ds(rbase + j * _ZRD, _ZRD)])

    plsc.subcore_barrier()

    crow = sid * _CPP
    pltpu.sync_copy(dst2_hbm.at[pl.ds(crow, _CPP)], idx_v)

    @pl.loop(0, _CPP // _NB)
    def _(g):
        for b in range(_NB):
            @pl.when(g > 0)
            def _():
                pltpu.make_async_copy(
                    ones_v, shared.at[idx_v.at[0]], ssem[b]).wait()

            pltpu.async_copy(
                ones_v, shared.at[idx_v.at[g * _NB + b]], ssem[b], add=True)

    for b in range(_NB):
        pltpu.make_async_copy(ones_v, shared.at[idx_v.at[0]], ssem[b]).wait()

    plsc.subcore_barrier()

    base = (cid * _NS + sid) * _GRW
    pltpu.sync_copy(shared.at[pl.ds(base, _GRW)], dbuf)

    @pl.loop(0, _GRW)
    def _(r):
        d = jnp.maximum(dbuf[r], 1.0)
        i = lax.bitcast_convert_type(d, jnp.int32)
        i = jnp.int32(0x5F3759DF) - lax.shift_right_logical(i, 1)
        y = lax.bitcast_convert_type(i, jnp.float32)
        y = y * (1.5 - 0.5 * d * y * y)
        y = y * (1.5 - 0.5 * d * y * y)
        y = y * (1.5 - 0.5 * d * y * y)
        nbuf[r] = y

    pltpu.sync_copy(nbuf, normb_hbm.at[pl.ds(base, _GRW)])

    for t in range(_GRW // _FT):
        row0 = base + t * _FT

        @pl.when(row0 < _N)
        def _():
            pltpu.sync_copy(feat_hbm.at[pl.ds(row0, _FT)], fbuf)

            @pl.loop(0, _FT)
            def _(r):
                nv = nbuf[t * _FT + r]
                for j in range(_D // 16):
                    gbuf[r, pl.ds(j * 16, 16)] = (
                        fbuf[r, pl.ds(j * 16, 16)] * nv)

            pltpu.sync_copy(gbuf, g_hbm.at[pl.ds(row0, _FT)])


def _agg_body(g_hbm, src2_hbm, dst2_hbm, aggp_hbm,
              shared, sidx_v, didx_v, r0, r1, r2, r3, r4, zer_v,
              g0, g1, g2, g3, g4, t0, t1, t2, t3, t4):
    cid = lax.axis_index("c")
    sid = lax.axis_index("s")
    rows = (r0, r1, r2, r3, r4)
    gsem = (g0, g1, g2, g3, g4)
    ssem = (t0, t1, t2, t3, t4)

    @pl.loop(0, _ZRA)
    def _(i):
        @pl.loop(0, _D // 16)
        def _(j):
            zer_v[i, pl.ds(j * 16, 16)] = jnp.zeros((16,), jnp.float32)

    rbase = sid * _RPT

    @pl.loop(0, _RPT // _ZRA)
    def _(j):
        pltpu.sync_copy(zer_v, shared.at[pl.ds(rbase + j * _ZRA, _ZRA)])

    plsc.subcore_barrier()

    crow = (sid * _NC + cid) * _CPT
    pltpu.sync_copy(src2_hbm.at[pl.ds(crow, _CPT)], sidx_v)
    pltpu.sync_copy(dst2_hbm.at[pl.ds(crow, _CPT)], didx_v)

    def issue_gather(t, b):
        pltpu.async_copy(g_hbm.at[sidx_v.at[t]], rows[b], gsem[b])

    def wait_gather(t, b):
        pltpu.make_async_copy(g_hbm.at[sidx_v.at[t]], rows[b], gsem[b]).wait()

    def issue_scatter(t, b):
        pltpu.async_copy(rows[b], shared.at[didx_v.at[t]], ssem[b], add=True)

    def wait_scatter(b):
        pltpu.make_async_copy(rows[b], shared.at[didx_v.at[0]], ssem[b]).wait()

    for b in range(_NB - 1):
        issue_gather(b, b)

    wait_gather(0, 0)
    issue_scatter(0, 0)
    issue_gather(_NB - 1, _NB - 1)
    for u in range(1, _NB):
        b = u % _NB
        wait_gather(u, b)
        issue_scatter(u, b)
        pb = (b + _NB - 1) % _NB
        wait_scatter(pb)
        issue_gather(u + _NB - 1, pb)

    @pl.loop(0, (_CPT - _NB) // _NB)
    def _(gg):
        for b in range(_NB):
            u = _NB + gg * _NB + b
            wait_gather(u, b)
            issue_scatter(u, b)
            pb = (b + _NB - 1) % _NB
            wait_scatter(pb)

            @pl.when(u + _NB - 1 <= _CPT - 1)
            def _():
                issue_gather(u + _NB - 1, pb)

    wait_scatter((_CPT - 1) % _NB)

    plsc.subcore_barrier()
    pltpu.sync_copy(shared.at[pl.ds(rbase, _RPT)],
                    aggp_hbm.at[cid].at[pl.ds(rbase, _RPT)])


def _final_body(aggp_ref, normb_ref, f0_ref, w_ref, b_ref, out_ref):
    norm = normb_ref[0:_N, 0:1]
    agg = aggp_ref[0, 0:_N, :] + aggp_ref[1, 0:_N, :]
    rst0 = agg * norm * (1.0 - _ALPHA) + _ALPHA * f0_ref[...]
    rst = (1.0 - _BETA) * rst0 + _BETA * jnp.dot(
        rst0, w_ref[...], preferred_element_type=jnp.float32)
    out_ref[...] = rst + b_ref[...]


_prep_call = functools.partial(
    pl.kernel,
    out_type=(
        jax.ShapeDtypeStruct((_NP, _D), jnp.float32),
        jax.ShapeDtypeStruct((_NP, 16), jnp.float32),
    ),
    mesh=_mesh,
    compiler_params=_sc_params,
    scratch_types=[
        pltpu.VMEM_SHARED((_NP, 16), jnp.float32),
        pltpu.VMEM((_CPP, _CH), jnp.int32),
        pltpu.VMEM((_CH, 16), jnp.float32),
        pltpu.VMEM((_ZRD, 16), jnp.float32),
        pltpu.VMEM((_GRW, 16), jnp.float32),
        pltpu.VMEM((_GRW, 16), jnp.float32),
        pltpu.VMEM((_FT, _D), jnp.float32),
        pltpu.VMEM((_FT, _D), jnp.float32),
    ] + [pltpu.SemaphoreType.DMA] * _NB,
)(_prep_body)


_agg_call = functools.partial(
    pl.kernel,
    out_type=jax.ShapeDtypeStruct((_NC, _NP, _D), jnp.float32),
    mesh=_mesh,
    compiler_params=_sc_params,
    scratch_types=[
        pltpu.VMEM_SHARED((_NP, _D), jnp.float32),
        pltpu.VMEM((_CPT, _CH), jnp.int32),
        pltpu.VMEM((_CPT, _CH), jnp.int32),
    ] + [pltpu.VMEM((_CH, _D), jnp.float32)] * _NB + [
        pltpu.VMEM((_ZRA, _D), jnp.float32),
    ] + [pltpu.SemaphoreType.DMA] * (2 * _NB),
)(_agg_body)


def kernel(feat, feat_0, edge_index, weight1, bias):
    src2 = edge_index[0].reshape(_NCH, _CH)
    dst2 = edge_index[1].reshape(_NCH, _CH)

    g, normb = _prep_call(dst2, feat)
    aggp = _agg_call(g, src2, dst2)

    out = pl.pallas_call(
        _final_body,
        out_shape=jax.ShapeDtypeStruct((_N, _D), jnp.float32),
    )(aggp, normb, feat_0, weight1, bias.reshape(1, _D))
    return out


# speedup vs baseline: 1.1550x; 1.0353x over previous
import functools
import math

import jax
import jax.numpy as jnp
from jax import lax
from jax.experimental import pallas as pl
from jax.experimental.pallas import tpu as pltpu
from jax.experimental.pallas import tpu_sc as plsc

_N = 10000
_E = 320000
_D = 128
_ALPHA = 0.1
_BETA = math.log(1.0 / 2.0 + 1.0)

_NC = 2
_NS = 16
_NW = _NC * _NS
_CH = 40
_NCH = _E // _CH
_CPT = _NCH // _NW
_NB = 5
_NP = 10240
_RPT = _NP // _NS
_ZRD = 128
_ZRA = 16
_CPP = _NCH // _NS
_GRW = _NP // _NW
_FT = 80
_CHA = 80
_NCHA = _E // _CHA
_CPA = _NCHA // _NW
_NBA = 2

_mesh = plsc.VectorSubcoreMesh(core_axis_name="c", subcore_axis_name="s")
_sc_params = pltpu.CompilerParams(use_tc_tiling_on_sc=False)


def _prep_body(dst2_hbm, feat_hbm, g_hbm, normb_hbm,
               shared, idx_v, ones_v, zer_v, dbuf, nbuf,
               f0b, f1b, f2b, f3b, g0b, g1b, g2b, g3b,
               s0, s1, s2, s3, s4, f0s, f1s, f2s, f3s, w0s, w1s, w2s, w3s):
    cid = lax.axis_index("c")
    sid = lax.axis_index("s")
    ssem = (s0, s1, s2, s3, s4)
    fbufs = (f0b, f1b, f2b, f3b)
    gbufs = (g0b, g1b, g2b, g3b)
    fsem = (f0s, f1s, f2s, f3s)
    wsem = (w0s, w1s, w2s, w3s)
    ntile = _GRW // _FT

    base = (cid * _NS + sid) * _GRW
    for t in range(ntile):
        @pl.when(base + t * _FT < _N)
        def _():
            pltpu.async_copy(
                feat_hbm.at[pl.ds(base + t * _FT, _FT)], fbufs[t], fsem[t])

    @pl.loop(0, _ZRD)
    def _(i):
        zer_v[i] = jnp.zeros((16,), jnp.float32)

    @pl.loop(0, _CH)
    def _(i):
        ones_v[i] = jnp.ones((16,), jnp.float32)

    rbase = sid * _RPT

    @pl.loop(0, _RPT // _ZRD)
    def _(j):
        pltpu.sync_copy(zer_v, shared.at[pl.ds(rbase + j * _ZRD, _ZRD)])

    plsc.subcore_barrier()

    crow = sid * _CPP
    pltpu.sync_copy(dst2_hbm.at[pl.ds(crow, _CPP)], idx_v)

    @pl.loop(0, _CPP // _NB)
    def _(g):
        for b in range(_NB):
            @pl.when(g > 0)
            def _():
                pltpu.make_async_copy(
                    ones_v, shared.at[idx_v.at[0]], ssem[b]).wait()

            pltpu.async_copy(
                ones_v, shared.at[idx_v.at[g * _NB + b]], ssem[b], add=True)

    for b in range(_NB):
        pltpu.make_async_copy(ones_v, shared.at[idx_v.at[0]], ssem[b]).wait()

    plsc.subcore_barrier()

    pltpu.sync_copy(shared.at[pl.ds(base, _GRW)], dbuf)

    @pl.loop(0, _GRW)
    def _(r):
        d = jnp.maximum(dbuf[r], 1.0)
        i = lax.bitcast_convert_type(d, jnp.int32)
        i = jnp.int32(0x5F3759DF) - lax.shift_right_logical(i, 1)
        y = lax.bitcast_convert_type(i, jnp.float32)
        y = y * (1.5 - 0.5 * d * y * y)
        y = y * (1.5 - 0.5 * d * y * y)
        y = y * (1.5 - 0.5 * d * y * y)
        nbuf[r] = y

    pltpu.sync_copy(nbuf, normb_hbm.at[pl.ds(base, _GRW)])

    for t in range(ntile):
        row0 = base + t * _FT

        @pl.when(row0 < _N)
        def _():
            pltpu.make_async_copy(
                feat_hbm.at[pl.ds(row0, _FT)], fbufs[t], fsem[t]).wait()

            @pl.loop(0, _FT)
            def _(r):
                nv = nbuf[t * _FT + r]
                for j in range(_D // 16):
                    gbufs[t][r, pl.ds(j * 16, 16)] = (
                        fbufs[t][r, pl.ds(j * 16, 16)] * nv)

            pltpu.async_copy(gbufs[t], g_hbm.at[pl.ds(row0, _FT)], wsem[t])

    for t in range(ntile):
        @pl.when(base + t * _FT < _N)
        def _():
            pltpu.make_async_copy(
                gbufs[t], g_hbm.at[pl.ds(base + t * _FT, _FT)],
                wsem[t]).wait()


def _agg_body(g_hbm, src2_hbm, dst2_hbm, aggp_hbm,
              shared, sidx_v, didx_v, r0, r1, r2, r3, r4, zer_v,
              g0, g1, g2, g3, g4, t0, t1, t2, t3, t4):
    cid = lax.axis_index("c")
    sid = lax.axis_index("s")
    rows = (r0, r1, r2, r3, r4)
    gsem = (g0, g1, g2, g3, g4)
    ssem = (t0, t1, t2, t3, t4)

    @pl.loop(0, _ZRA)
    def _(i):
        @pl.loop(0, _D // 16)
        def _(j):
            zer_v[i, pl.ds(j * 16, 16)] = jnp.zeros((16,), jnp.float32)

    rbase = sid * _RPT

    @pl.loop(0, _RPT // _ZRA)
    def _(j):
        pltpu.sync_copy(zer_v, shared.at[pl.ds(rbase + j * _ZRA, _ZRA)])

    plsc.subcore_barrier()

    crow = (sid * _NC + cid) * _CPT
    pltpu.sync_copy(src2_hbm.at[pl.ds(crow, _CPT)], sidx_v)
    pltpu.sync_copy(dst2_hbm.at[pl.ds(crow, _CPT)], didx_v)

    def issue_gather(t, b):
        pltpu.async_copy(g_hbm.at[sidx_v.at[t]], rows[b], gsem[b])

    def wait_gather(t, b):
        pltpu.make_async_copy(g_hbm.at[sidx_v.at[t]], rows[b], gsem[b]).wait()

    def issue_scatter(t, b):
        pltpu.async_copy(rows[b], shared.at[didx_v.at[t]], ssem[b], add=True)

    def wait_scatter(b):
        pltpu.make_async_copy(rows[b], shared.at[didx_v.at[0]], ssem[b]).wait()

    for b in range(_NB - 1):
        issue_gather(b, b)

    wait_gather(0, 0)
    issue_scatter(0, 0)
    issue_gather(_NB - 1, _NB - 1)
    for u in range(1, _NB):
        b = u % _NB
        wait_gather(u, b)
        issue_scatter(u, b)
        pb = (b + _NB - 1) % _NB
        wait_scatter(pb)
        issue_gather(u + _NB - 1, pb)

    @pl.loop(0, (_CPT - _NB) // _NB)
    def _(gg):
        for b in range(_NB):
            u = _NB + gg * _NB + b
            wait_gather(u, b)
            issue_scatter(u, b)
            pb = (b + _NB - 1) % _NB
            wait_scatter(pb)

            @pl.when(u + _NB - 1 <= _CPT - 1)
            def _():
                issue_gather(u + _NB - 1, pb)

    wait_scatter((_CPT - 1) % _NB)

    plsc.subcore_barrier()
    pltpu.sync_copy(shared.at[pl.ds(rbase, _RPT)],
                    aggp_hbm.at[cid].at[pl.ds(rbase, _RPT)])


def _final_body(aggp_ref, normb_ref, f0_ref, w_ref, b_ref, out_ref):
    norm = normb_ref[0:_N, 0:1]
    agg = aggp_ref[0, 0:_N, :] + aggp_ref[1, 0:_N, :]
    rst0 = agg * norm * (1.0 - _ALPHA) + _ALPHA * f0_ref[...]
    rst = (1.0 - _BETA) * rst0 + _BETA * jnp.dot(
        rst0, w_ref[...], preferred_element_type=jnp.float32)
    out_ref[...] = rst + b_ref[...]


_prep_call = functools.partial(
    pl.kernel,
    out_type=(
        jax.ShapeDtypeStruct((_NP, _D), jnp.float32),
        jax.ShapeDtypeStruct((_NP, 16), jnp.float32),
    ),
    mesh=_mesh,
    compiler_params=_sc_params,
    scratch_types=[
        pltpu.VMEM_SHARED((_NP, 16), jnp.float32),
        pltpu.VMEM((_CPP, _CH), jnp.int32),
        pltpu.VMEM((_CH, 16), jnp.float32),
        pltpu.VMEM((_ZRD, 16), jnp.float32),
        pltpu.VMEM((_GRW, 16), jnp.float32),
        pltpu.VMEM((_GRW, 16), jnp.float32),
    ] + [pltpu.VMEM((_FT, _D), jnp.float32)] * (2 * (_GRW // _FT))
    + [pltpu.SemaphoreType.DMA] * (_NB + 2 * (_GRW // _FT)),
)(_prep_body)


_agg_call = functools.partial(
    pl.kernel,
    out_type=jax.ShapeDtypeStruct((_NC, _NP, _D), jnp.float32),
    mesh=_mesh,
    compiler_params=_sc_params,
    scratch_types=[
        pltpu.VMEM_SHARED((_NP, _D), jnp.float32),
        pltpu.VMEM((_CPT, _CH), jnp.int32),
        pltpu.VMEM((_CPT, _CH), jnp.int32),
    ] + [pltpu.VMEM((_CH, _D), jnp.float32)] * _NB + [
        pltpu.VMEM((_ZRA, _D), jnp.float32),
    ] + [pltpu.SemaphoreType.DMA] * (2 * _NB),
)(_agg_body)


def kernel(feat, feat_0, edge_index, weight1, bias):
    src2 = edge_index[0].reshape(_NCH, _CH)
    dst2 = edge_index[1].reshape(_NCH, _CH)

    g, normb = _prep_call(dst2, feat)
    aggp = _agg_call(g, src2, dst2)

    out = pl.pallas_call(
        _final_body,
        out_shape=jax.ShapeDtypeStruct((_N, _D), jnp.float32),
    )(aggp, normb, feat_0, weight1, bias.reshape(1, _D))
    return out


# speedup vs baseline: 1.1695x; 1.0126x over previous
import functools
import math

import jax
import jax.numpy as jnp
from jax import lax
from jax.experimental import pallas as pl
from jax.experimental.pallas import tpu as pltpu
from jax.experimental.pallas import tpu_sc as plsc

_N = 10000
_E = 320000
_D = 128
_ALPHA = 0.1
_BETA = math.log(1.0 / 2.0 + 1.0)

_NC = 2
_NS = 16
_NW = _NC * _NS
_CH = 40
_NCH = _E // _CH
_CPT = _NCH // _NW
_NB = 5
_NP = 10240
_RPT = _NP // _NS
_ZRD = 128
_ZRA = 16
_CPP = _NCH // _NS
_GRW = _NP // _NW
_FT = 80

_mesh = plsc.VectorSubcoreMesh(core_axis_name="c", subcore_axis_name="s")
_sc_params = pltpu.CompilerParams(use_tc_tiling_on_sc=False)


def _prep_body(dst2_hbm, feat_hbm, g_hbm, normb_hbm,
               shared, idx_v, ones_v, zer_v, dbuf, nbuf,
               f0b, f1b, f2b, f3b, g0b, g1b, g2b, g3b,
               s0, s1, s2, s3, s4, f0s, f1s, f2s, f3s, w0s, w1s, w2s, w3s):
    cid = lax.axis_index("c")
    sid = lax.axis_index("s")
    ssem = (s0, s1, s2, s3, s4)
    fbufs = (f0b, f1b, f2b, f3b)
    gbufs = (g0b, g1b, g2b, g3b)
    fsem = (f0s, f1s, f2s, f3s)
    wsem = (w0s, w1s, w2s, w3s)
    ntile = _GRW // _FT

    base = (cid * _NS + sid) * _GRW
    for t in range(ntile):
        @pl.when(base + t * _FT < _N)
        def _():
            pltpu.async_copy(
                feat_hbm.at[pl.ds(base + t * _FT, _FT)], fbufs[t], fsem[t])

    @pl.loop(0, _ZRD)
    def _(i):
        zer_v[i] = jnp.zeros((16,), jnp.float32)

    @pl.loop(0, _CH)
    def _(i):
        ones_v[i] = jnp.ones((16,), jnp.float32)

    rbase = sid * _RPT

    @pl.loop(0, _RPT // _ZRD)
    def _(j):
        pltpu.sync_copy(zer_v, shared.at[pl.ds(rbase + j * _ZRD, _ZRD)])

    plsc.subcore_barrier()

    crow = sid * _CPP
    pltpu.sync_copy(dst2_hbm.at[pl.ds(crow, _CPP)], idx_v)

    @pl.loop(0, _CPP // _NB)
    def _(g):
        for b in range(_NB):
            @pl.when(g > 0)
            def _():
                pltpu.make_async_copy(
                    ones_v, shared.at[idx_v.at[0]], ssem[b]).wait()

            pltpu.async_copy(
                ones_v, shared.at[idx_v.at[g * _NB + b]], ssem[b], add=True)

    for b in range(_NB):
        pltpu.make_async_copy(ones_v, shared.at[idx_v.at[0]], ssem[b]).wait()

    plsc.subcore_barrier()

    pltpu.sync_copy(shared.at[pl.ds(base, _GRW)], dbuf)

    @pl.loop(0, _GRW)
    def _(r):
        d = jnp.maximum(dbuf[r], 1.0)
        i = lax.bitcast_convert_type(d, jnp.int32)
        i = jnp.int32(0x5F3759DF) - lax.shift_right_logical(i, 1)
        y = lax.bitcast_convert_type(i, jnp.float32)
        y = y * (1.5 - 0.5 * d * y * y)
        y = y * (1.5 - 0.5 * d * y * y)
        y = y * (1.5 - 0.5 * d * y * y)
        nbuf[r] = y

    pltpu.sync_copy(nbuf, normb_hbm.at[pl.ds(base, _GRW)])

    for t in range(ntile):
        row0 = base + t * _FT

        @pl.when(row0 < _N)
        def _():
            pltpu.make_async_copy(
                feat_hbm.at[pl.ds(row0, _FT)], fbufs[t], fsem[t]).wait()

            @pl.loop(0, _FT)
            def _(r):
                nv = nbuf[t * _FT + r]
                for j in range(_D // 16):
                    gbufs[t][r, pl.ds(j * 16, 16)] = (
                        fbufs[t][r, pl.ds(j * 16, 16)] * nv)

            pltpu.async_copy(gbufs[t], g_hbm.at[pl.ds(row0, _FT)], wsem[t])

    for t in range(ntile):
        @pl.when(base + t * _FT < _N)
        def _():
            pltpu.make_async_copy(
                gbufs[t], g_hbm.at[pl.ds(base + t * _FT, _FT)],
                wsem[t]).wait()


def _agg_body(g_hbm, src2_hbm, dst2_hbm, aggp_hbm,
              shared, sidx_v, didx_v, r0, r1, r2, r3, r4, zer_v,
              g0, g1, g2, g3, g4, t0, t1, t2, t3, t4):
    cid = lax.axis_index("c")
    sid = lax.axis_index("s")
    rows = (r0, r1, r2, r3, r4)
    gsem = (g0, g1, g2, g3, g4)
    ssem = (t0, t1, t2, t3, t4)

    @pl.loop(0, _ZRA)
    def _(i):
        @pl.loop(0, _D // 16)
        def _(j):
            zer_v[i, pl.ds(j * 16, 16)] = jnp.zeros((16,), jnp.float32)

    crow = (sid * _NC + cid) * _CPT
    pltpu.async_copy(src2_hbm.at[pl.ds(crow, _CPT)], sidx_v, gsem[0])
    pltpu.async_copy(dst2_hbm.at[pl.ds(crow, _CPT)], didx_v, gsem[1])

    rbase = sid * _RPT

    @pl.loop(0, _RPT // _ZRA)
    def _(j):
        pltpu.sync_copy(zer_v, shared.at[pl.ds(rbase + j * _ZRA, _ZRA)])

    pltpu.make_async_copy(src2_hbm.at[pl.ds(crow, _CPT)], sidx_v,
                          gsem[0]).wait()
    pltpu.make_async_copy(dst2_hbm.at[pl.ds(crow, _CPT)], didx_v,
                          gsem[1]).wait()
    plsc.subcore_barrier()

    def issue_gather(t, b):
        pltpu.async_copy(g_hbm.at[sidx_v.at[t]], rows[b], gsem[b])

    def wait_gather(t, b):
        pltpu.make_async_copy(g_hbm.at[sidx_v.at[t]], rows[b], gsem[b]).wait()

    def issue_scatter(t, b):
        pltpu.async_copy(rows[b], shared.at[didx_v.at[t]], ssem[b], add=True)

    def wait_scatter(b):
        pltpu.make_async_copy(rows[b], shared.at[didx_v.at[0]], ssem[b]).wait()

    for b in range(_NB - 1):
        issue_gather(b, b)

    wait_gather(0, 0)
    issue_scatter(0, 0)
    issue_gather(_NB - 1, _NB - 1)
    for u in range(1, _NB):
        b = u % _NB
        wait_gather(u, b)
        issue_scatter(u, b)
        pb = (b + _NB - 1) % _NB
        wait_scatter(pb)
        issue_gather(u + _NB - 1, pb)

    @pl.loop(0, (_CPT - _NB) // _NB)
    def _(gg):
        for b in range(_NB):
            u = _NB + gg * _NB + b
            wait_gather(u, b)
            issue_scatter(u, b)
            pb = (b + _NB - 1) % _NB
            wait_scatter(pb)

            @pl.when(u + _NB - 1 <= _CPT - 1)
            def _():
                issue_gather(u + _NB - 1, pb)

    wait_scatter((_CPT - 1) % _NB)

    plsc.subcore_barrier()
    pltpu.sync_copy(shared.at[pl.ds(rbase, _RPT)],
                    aggp_hbm.at[cid].at[pl.ds(rbase, _RPT)])


def _final_body(aggp_ref, normb_ref, f0_ref, w_ref, b_ref, out_ref):
    norm = normb_ref[0:_N, 0:1]
    agg = aggp_ref[0, 0:_N, :] + aggp_ref[1, 0:_N, :]
    rst0 = agg * norm * (1.0 - _ALPHA) + _ALPHA * f0_ref[...]
    rst = (1.0 - _BETA) * rst0 + _BETA * jnp.dot(
        rst0, w_ref[...], preferred_element_type=jnp.float32)
    out_ref[...] = rst + b_ref[...]


_prep_call = functools.partial(
    pl.kernel,
    out_type=(
        jax.ShapeDtypeStruct((_NP, _D), jnp.float32),
        jax.ShapeDtypeStruct((_NP, 16), jnp.float32),
    ),
    mesh=_mesh,
    compiler_params=_sc_params,
    scratch_types=[
        pltpu.VMEM_SHARED((_NP, 16), jnp.float32),
        pltpu.VMEM((_CPP, _CH), jnp.int32),
        pltpu.VMEM((_CH, 16), jnp.float32),
        pltpu.VMEM((_ZRD, 16), jnp.float32),
        pltpu.VMEM((_GRW, 16), jnp.float32),
        pltpu.VMEM((_GRW, 16), jnp.float32),
    ] + [pltpu.VMEM((_FT, _D), jnp.float32)] * (2 * (_GRW // _FT))
    + [pltpu.SemaphoreType.DMA] * (_NB + 2 * (_GRW // _FT)),
)(_prep_body)


_agg_call = functools.partial(
    pl.kernel,
    out_type=jax.ShapeDtypeStruct((_NC, _NP, _D), jnp.float32),
    mesh=_mesh,
    compiler_params=_sc_params,
    scratch_types=[
        pltpu.VMEM_SHARED((_NP, _D), jnp.float32),
        pltpu.VMEM((_CPT, _CH), jnp.int32),
        pltpu.VMEM((_CPT, _CH), jnp.int32),
    ] + [pltpu.VMEM((_CH, _D), jnp.float32)] * _NB + [
        pltpu.VMEM((_ZRA, _D), jnp.float32),
    ] + [pltpu.SemaphoreType.DMA] * (2 * _NB),
)(_agg_body)


def kernel(feat, feat_0, edge_index, weight1, bias):
    src2 = edge_index[0].reshape(_NCH, _CH)
    dst2 = edge_index[1].reshape(_NCH, _CH)

    g, normb = _prep_call(dst2, feat)
    aggp = _agg_call(g, src2, dst2)

    out = pl.pallas_call(
        _final_body,
        out_shape=jax.ShapeDtypeStruct((_N, _D), jnp.float32),
    )(aggp, normb, feat_0, weight1, bias.reshape(1, _D))
    return out
